# Initial kernel scaffold; baseline (speedup 1.0000x reference)
import functools

import jax
import jax.numpy as jnp
from jax import lax
from jax.experimental import pallas as pl
from jax.experimental.pallas import tpu as pltpu
from jax.experimental.pallas import tpu_sc as plsc

N = 10000
E = 160000
DF = 256
DE = 16

NUM_CORES = 2
NUM_SUBCORES = 16
NUM_TILES = NUM_CORES * NUM_SUBCORES
EDGES_PER_TILE = E // NUM_TILES
CHUNK = 125
CHUNKS_PER_TILE = EDGES_PER_TILE // CHUNK
ROWS_PER_SUBCORE = N // NUM_SUBCORES


def _sc_segment_sum(recv2d, edge_attr, zeros_init):
    mesh = plsc.VectorSubcoreMesh(core_axis_name="c", subcore_axis_name="s")

    @functools.partial(
        pl.kernel,
        mesh=mesh,
        out_type=jax.ShapeDtypeStruct((NUM_CORES, N, DE), jnp.float32),
        scratch_types=[
            pltpu.VMEM((CHUNKS_PER_TILE, CHUNK), jnp.int32),
            pltpu.VMEM((EDGES_PER_TILE, DE), jnp.float32),
            pltpu.VMEM_SHARED((N, DE), jnp.float32),
        ],
    )
    def sc_kernel(recv_hbm, attr_hbm, zeros_hbm, out_hbm, idx_v, rows_v, aggr_sh):
        c = lax.axis_index("c")
        s = lax.axis_index("s")
        w = c * NUM_SUBCORES + s

        row0 = s * ROWS_PER_SUBCORE
        pltpu.sync_copy(
            zeros_hbm.at[pl.ds(row0, ROWS_PER_SUBCORE)],
            aggr_sh.at[pl.ds(row0, ROWS_PER_SUBCORE)],
        )

        pltpu.sync_copy(recv_hbm.at[pl.ds(w * CHUNKS_PER_TILE, CHUNKS_PER_TILE)], idx_v)
        pltpu.sync_copy(attr_hbm.at[pl.ds(w * EDGES_PER_TILE, EDGES_PER_TILE)], rows_v)

        plsc.subcore_barrier()

        def body(j, carry):
            pltpu.sync_copy(
                rows_v.at[pl.ds(j * CHUNK, CHUNK)],
                aggr_sh.at[idx_v.at[j]],
                add=True,
            )
            return carry

        lax.fori_loop(0, CHUNKS_PER_TILE, body, 0)

        plsc.subcore_barrier()

        pltpu.sync_copy(
            aggr_sh.at[pl.ds(row0, ROWS_PER_SUBCORE)],
            out_hbm.at[c, pl.ds(row0, ROWS_PER_SUBCORE), :],
        )

    return sc_kernel(recv2d, edge_attr, zeros_init)


ROW_BLOCK = 2000


def _tc_body(x_ref, p_ref, w1_ref, w2_ref, b_ref, o_ref):
    aggr = p_ref[0] + p_ref[1]
    o_ref[...] = (
        jnp.dot(x_ref[...], w1_ref[...], preferred_element_type=jnp.float32)
        + jnp.dot(aggr, w2_ref[...], preferred_element_type=jnp.float32)
        + b_ref[...]
    )


def _tc_update(x, partials, W1, W2, b2d):
    grid = N // ROW_BLOCK
    return pl.pallas_call(
        _tc_body,
        grid=(grid,),
        in_specs=[
            pl.BlockSpec((ROW_BLOCK, DF), lambda i: (i, 0)),
            pl.BlockSpec((NUM_CORES, ROW_BLOCK, DE), lambda i: (0, i, 0)),
            pl.BlockSpec((DF, DF), lambda i: (0, 0)),
            pl.BlockSpec((DE, DF), lambda i: (0, 0)),
            pl.BlockSpec((1, DF), lambda i: (0, 0)),
        ],
        out_specs=pl.BlockSpec((ROW_BLOCK, DF), lambda i: (i, 0)),
        out_shape=jax.ShapeDtypeStruct((N, DF), jnp.float32),
    )(x, partials, W1, W2, b2d)


def kernel(x, edge_index, edge_attr, pos, W, b):
    recv2d = edge_index[1].reshape(NUM_TILES * CHUNKS_PER_TILE, CHUNK)
    zeros_init = jnp.zeros((N, DE), jnp.float32)
    partials = _sc_segment_sum(recv2d, edge_attr, zeros_init)
    W1 = W[:DF]
    W2 = W[DF:]
    b2d = b.reshape(1, DF)
    return _tc_update(x, partials, W1, W2, b2d)


# speedup vs baseline: 4.6843x; 4.6843x over previous
import functools

import jax
import jax.numpy as jnp
from jax import lax
from jax.experimental import pallas as pl
from jax.experimental.pallas import tpu as pltpu
from jax.experimental.pallas import tpu_sc as plsc

N = 10000
E = 160000
DF = 256
DE = 16

NUM_CORES = 2
NUM_SUBCORES = 16
NUM_TILES = NUM_CORES * NUM_SUBCORES
EDGES_PER_TILE = E // NUM_TILES
CHUNK = 125
CHUNKS_PER_TILE = EDGES_PER_TILE // CHUNK
NPAD = 10240
ROWS_PER_SUBCORE = NPAD // NUM_SUBCORES


def _sc_segment_sum(recv2d, edge_attr, zeros_init):
    mesh = plsc.VectorSubcoreMesh(core_axis_name="c", subcore_axis_name="s")

    @functools.partial(
        pl.kernel,
        mesh=mesh,
        compiler_params=pltpu.CompilerParams(use_tc_tiling_on_sc=False),
        out_type=jax.ShapeDtypeStruct((NUM_CORES, NPAD, DE), jnp.float32),
        scratch_types=[
            pltpu.VMEM((CHUNKS_PER_TILE, CHUNK), jnp.int32),
            pltpu.VMEM((EDGES_PER_TILE, DE), jnp.float32),
            pltpu.VMEM_SHARED((NPAD, DE), jnp.float32),
        ],
    )
    def sc_kernel(recv_hbm, attr_hbm, zeros_hbm, out_hbm, idx_v, rows_v, aggr_sh):
        c = lax.axis_index("c")
        s = lax.axis_index("s")
        w = c * NUM_SUBCORES + s

        row0 = s * ROWS_PER_SUBCORE
        pltpu.sync_copy(
            zeros_hbm.at[pl.ds(row0, ROWS_PER_SUBCORE)],
            aggr_sh.at[pl.ds(row0, ROWS_PER_SUBCORE)],
        )

        pltpu.sync_copy(recv_hbm.at[pl.ds(w * CHUNKS_PER_TILE, CHUNKS_PER_TILE)], idx_v)
        pltpu.sync_copy(attr_hbm.at[pl.ds(w * EDGES_PER_TILE, EDGES_PER_TILE)], rows_v)

        plsc.subcore_barrier()

        def body(j, carry):
            pltpu.sync_copy(
                rows_v.at[pl.ds(j * CHUNK, CHUNK)],
                aggr_sh.at[idx_v.at[j]],
                add=True,
            )
            return carry

        lax.fori_loop(0, CHUNKS_PER_TILE, body, 0)

        plsc.subcore_barrier()

        pltpu.sync_copy(
            aggr_sh.at[pl.ds(row0, ROWS_PER_SUBCORE)],
            out_hbm.at[c, pl.ds(row0, ROWS_PER_SUBCORE), :],
        )

    return sc_kernel(recv2d, edge_attr, zeros_init)


ROW_BLOCK = 2000


def _tc_body(x_ref, p_ref, w1_ref, w2_ref, b_ref, o_ref):
    aggr = p_ref[0] + p_ref[1]
    o_ref[...] = (
        jnp.dot(x_ref[...], w1_ref[...], preferred_element_type=jnp.float32)
        + jnp.dot(aggr, w2_ref[...], preferred_element_type=jnp.float32)
        + b_ref[...]
    )


def _tc_update(x, partials, W1, W2, b2d):
    grid = N // ROW_BLOCK
    return pl.pallas_call(
        _tc_body,
        grid=(grid,),
        in_specs=[
            pl.BlockSpec((ROW_BLOCK, DF), lambda i: (i, 0)),
            pl.BlockSpec((NUM_CORES, ROW_BLOCK, DE), lambda i: (0, i, 0)),
            pl.BlockSpec((DF, DF), lambda i: (0, 0)),
            pl.BlockSpec((DE, DF), lambda i: (0, 0)),
            pl.BlockSpec((1, DF), lambda i: (0, 0)),
        ],
        out_specs=pl.BlockSpec((ROW_BLOCK, DF), lambda i: (i, 0)),
        out_shape=jax.ShapeDtypeStruct((N, DF), jnp.float32),
    )(x, partials, W1, W2, b2d)


def kernel(x, edge_index, edge_attr, pos, W, b):
    recv2d = edge_index[1].reshape(NUM_TILES * CHUNKS_PER_TILE, CHUNK)
    zeros_init = jnp.zeros((NPAD, DE), jnp.float32)
    partials = _sc_segment_sum(recv2d, edge_attr, zeros_init)
    W1 = W[:DF]
    W2 = W[DF:]
    b2d = b.reshape(1, DF)
    return _tc_update(x, partials, W1, W2, b2d)


# speedup vs baseline: 6.6225x; 1.4138x over previous
import functools

import jax
import jax.numpy as jnp
from jax import lax
from jax.experimental import pallas as pl
from jax.experimental.pallas import tpu as pltpu
from jax.experimental.pallas import tpu_sc as plsc

N = 10000
E = 160000
DF = 256
DE = 16

NUM_CORES = 2
NUM_SUBCORES = 16
NUM_TILES = NUM_CORES * NUM_SUBCORES
EB = E // 128
EB_SMALL = EB // NUM_TILES
EB_BIG = EB_SMALL + 1
NPAD = 10240
ROWS_PER_SUBCORE = NPAD // NUM_SUBCORES


def _sc_segment_sum(recv, q, zeros_init):
    mesh = plsc.VectorSubcoreMesh(core_axis_name="c", subcore_axis_name="s")

    @functools.partial(
        pl.kernel,
        mesh=mesh,
        compiler_params=pltpu.CompilerParams(
            use_tc_tiling_on_sc=False, needs_layout_passes=False
        ),
        out_type=jax.ShapeDtypeStruct((NUM_CORES, NPAD, DE), jnp.float32),
        scratch_types=[
            pltpu.VMEM((EB_BIG * 128,), jnp.int32),
            pltpu.VMEM((2, EB_BIG, 8, 128), jnp.float32),
            pltpu.VMEM((128, DE), jnp.float32),
            pltpu.VMEM_SHARED((NPAD, DE), jnp.float32),
        ],
    )
    def sc_kernel(recv_hbm, q_hbm, zeros_hbm, out_hbm, idx_v, slab_v, tbuf, aggr_sh):
        c = lax.axis_index("c")
        s = lax.axis_index("s")
        w = c * NUM_SUBCORES + s
        n_extra = EB % NUM_TILES
        base_eb = w * EB_SMALL + jnp.minimum(w, n_extra)

        row0 = s * ROWS_PER_SUBCORE
        pltpu.sync_copy(
            zeros_hbm.at[pl.ds(row0, ROWS_PER_SUBCORE)],
            aggr_sh.at[pl.ds(row0, ROWS_PER_SUBCORE)],
        )

        @pl.when(w < n_extra)
        def _():
            pltpu.sync_copy(
                recv_hbm.at[pl.ds(base_eb * 128, EB_BIG * 128)], idx_v
            )
            pltpu.sync_copy(q_hbm.at[:, pl.ds(base_eb, EB_BIG)], slab_v)

        @pl.when(w >= n_extra)
        def _():
            pltpu.sync_copy(
                recv_hbm.at[pl.ds(base_eb * 128, EB_SMALL * 128)],
                idx_v.at[pl.ds(0, EB_SMALL * 128)],
            )
            pltpu.sync_copy(
                q_hbm.at[:, pl.ds(base_eb, EB_SMALL)],
                slab_v.at[:, pl.ds(0, EB_SMALL)],
            )

        plsc.subcore_barrier()

        count = jnp.where(w < n_extra, EB_BIG, EB_SMALL)
        iota = lax.broadcasted_iota(jnp.int32, (16,), 0)

        def body(i, carry):
            for fb in range(2):
                for fi in range(8):
                    col = jnp.full((16,), fb * 8 + fi, jnp.int32)
                    for e16 in range(8):
                        v = slab_v[fb, i, fi, pl.ds(e16 * 16, 16)]
                        plsc.store_scatter(tbuf, [iota + (e16 * 16), col], v)
            pltpu.sync_copy(
                tbuf, aggr_sh.at[idx_v.at[pl.ds(i * 128, 128)]], add=True
            )
            return carry

        lax.fori_loop(0, count, body, 0)

        plsc.subcore_barrier()

        pltpu.sync_copy(
            aggr_sh.at[pl.ds(row0, ROWS_PER_SUBCORE)],
            out_hbm.at[c, pl.ds(row0, ROWS_PER_SUBCORE), :],
        )

    return sc_kernel(recv, q, zeros_init)


ROW_BLOCK = 2000


def _tc_body(x_ref, p_ref, w1_ref, w2_ref, b_ref, o_ref):
    aggr = p_ref[0] + p_ref[1]
    o_ref[...] = (
        jnp.dot(x_ref[...], w1_ref[...], preferred_element_type=jnp.float32)
        + jnp.dot(aggr, w2_ref[...], preferred_element_type=jnp.float32)
        + b_ref[...]
    )


def _tc_update(x, partials, W1, W2, b2d):
    grid = N // ROW_BLOCK
    return pl.pallas_call(
        _tc_body,
        grid=(grid,),
        in_specs=[
            pl.BlockSpec((ROW_BLOCK, DF), lambda i: (i, 0)),
            pl.BlockSpec((NUM_CORES, ROW_BLOCK, DE), lambda i: (0, i, 0)),
            pl.BlockSpec((DF, DF), lambda i: (0, 0)),
            pl.BlockSpec((DE, DF), lambda i: (0, 0)),
            pl.BlockSpec((1, DF), lambda i: (0, 0)),
        ],
        out_specs=pl.BlockSpec((ROW_BLOCK, DF), lambda i: (i, 0)),
        out_shape=jax.ShapeDtypeStruct((N, DF), jnp.float32),
    )(x, partials, W1, W2, b2d)


def kernel(x, edge_index, edge_attr, pos, W, b):
    recv = edge_index[1]
    q = edge_attr.reshape(EB, 128, 2, 8).transpose(2, 0, 3, 1)
    zeros_init = jnp.zeros((NPAD, DE), jnp.float32)
    partials = _sc_segment_sum(recv, q, zeros_init)
    W1 = W[:DF]
    W2 = W[DF:]
    b2d = b.reshape(1, DF)
    return _tc_update(x, partials, W1, W2, b2d)


# speedup vs baseline: 7.5435x; 1.1391x over previous
import functools

import jax
import jax.numpy as jnp
from jax import lax
from jax.experimental import pallas as pl
from jax.experimental.pallas import tpu as pltpu
from jax.experimental.pallas import tpu_sc as plsc

N = 10000
E = 160000
DF = 256
DE = 16

NUM_CORES = 2
NUM_SUBCORES = 16
NUM_TILES = NUM_CORES * NUM_SUBCORES
EB = E // 128
EB_SMALL = EB // NUM_TILES
EB_BIG = EB_SMALL + 1
N_EXTRA = EB % NUM_TILES
NPAD = 10240
ROWS_PER_SUBCORE = NPAD // NUM_SUBCORES
SCATTER_BYTES = 128 * DE * 4


def _sc_segment_sum(recv3, q, zeros_init):
    mesh = plsc.VectorSubcoreMesh(core_axis_name="c", subcore_axis_name="s")

    @functools.partial(
        pl.kernel,
        mesh=mesh,
        compiler_params=pltpu.CompilerParams(
            use_tc_tiling_on_sc=False, needs_layout_passes=False
        ),
        out_type=jax.ShapeDtypeStruct((NUM_CORES, NPAD, DE), jnp.float32),
        scratch_types=[
            pltpu.VMEM((EB_BIG, 128), jnp.int32),
            pltpu.VMEM((2, EB_BIG, 8, 128), jnp.float32),
            pltpu.VMEM((128, DE), jnp.float32),
            pltpu.VMEM((128, DE), jnp.float32),
            pltpu.VMEM_SHARED((NPAD, DE), jnp.float32),
            pltpu.SemaphoreType.DMA,
        ],
    )
    def sc_kernel(
        recv_hbm, q_hbm, zeros_hbm, out_hbm, idx_v, slab_v, tb0, tb1, aggr_sh, sem
    ):
        c = lax.axis_index("c")
        s = lax.axis_index("s")
        w = c * NUM_SUBCORES + s
        base_eb = w * EB_SMALL + jnp.minimum(w, N_EXTRA)

        row0 = s * ROWS_PER_SUBCORE
        pltpu.sync_copy(
            zeros_hbm.at[pl.ds(row0, ROWS_PER_SUBCORE)],
            aggr_sh.at[pl.ds(row0, ROWS_PER_SUBCORE)],
        )

        @pl.when(w < N_EXTRA)
        def _():
            pltpu.sync_copy(recv_hbm.at[pl.ds(base_eb, EB_BIG), 1, :], idx_v)
            pltpu.sync_copy(q_hbm.at[:, pl.ds(base_eb, EB_BIG)], slab_v)

        @pl.when(w >= N_EXTRA)
        def _():
            pltpu.sync_copy(
                recv_hbm.at[pl.ds(base_eb, EB_SMALL), 1, :],
                idx_v.at[pl.ds(0, EB_SMALL)],
            )
            pltpu.sync_copy(
                q_hbm.at[:, pl.ds(base_eb, EB_SMALL)],
                slab_v.at[:, pl.ds(0, EB_SMALL)],
            )

        plsc.subcore_barrier()

        count = jnp.where(w < N_EXTRA, EB_BIG, EB_SMALL)
        iota = lax.broadcasted_iota(jnp.int32, (16,), 0)

        def transpose_block(i, tbuf):
            for fb in range(2):
                for fi in range(8):
                    col = jnp.full((16,), fb * 8 + fi, jnp.int32)
                    for e16 in range(8):
                        v = slab_v[fb, i, fi, pl.ds(e16 * 16, 16)]
                        plsc.store_scatter(tbuf, [iota + (e16 * 16), col], v)

        def start_scatter(i, tbuf):
            pltpu.make_async_copy(tbuf, aggr_sh.at[idx_v.at[i]], sem).start(
                add=True
            )

        def drain_one():
            pltpu.make_async_copy(tb0, aggr_sh.at[pl.ds(0, 128)], sem).wait()

        def body(p, carry):
            i0 = 2 * p
            i1 = 2 * p + 1

            @pl.when(p > 0)
            def _():
                drain_one()

            transpose_block(i0, tb0)
            start_scatter(i0, tb0)

            @pl.when(i1 < count)
            def _():
                @pl.when(p > 0)
                def _():
                    drain_one()

                transpose_block(i1, tb1)
                start_scatter(i1, tb1)

            return carry

        n_pairs = (count + 1) // 2
        lax.fori_loop(0, n_pairs, body, 0)
        drain_one()
        drain_one()

        plsc.subcore_barrier()

        pltpu.sync_copy(
            aggr_sh.at[pl.ds(row0, ROWS_PER_SUBCORE)],
            out_hbm.at[c, pl.ds(row0, ROWS_PER_SUBCORE), :],
        )

    return sc_kernel(recv3, q, zeros_init)


ROW_BLOCK = 2000


def _tc_body(x_ref, p_ref, w1_ref, w2_ref, b_ref, o_ref):
    aggr = p_ref[0] + p_ref[1]
    o_ref[...] = (
        jnp.dot(x_ref[...], w1_ref[...], preferred_element_type=jnp.float32)
        + jnp.dot(aggr, w2_ref[...], preferred_element_type=jnp.float32)
        + b_ref[...]
    )


def _tc_update(x, partials, W1, W2, b2d):
    grid = N // ROW_BLOCK
    return pl.pallas_call(
        _tc_body,
        grid=(grid,),
        in_specs=[
            pl.BlockSpec((ROW_BLOCK, DF), lambda i: (i, 0)),
            pl.BlockSpec((NUM_CORES, ROW_BLOCK, DE), lambda i: (0, i, 0)),
            pl.BlockSpec((DF, DF), lambda i: (0, 0)),
            pl.BlockSpec((DE, DF), lambda i: (0, 0)),
            pl.BlockSpec((1, DF), lambda i: (0, 0)),
        ],
        out_specs=pl.BlockSpec((ROW_BLOCK, DF), lambda i: (i, 0)),
        out_shape=jax.ShapeDtypeStruct((N, DF), jnp.float32),
    )(x, partials, W1, W2, b2d)


def kernel(x, edge_index, edge_attr, pos, W, b):
    recv3 = edge_index.reshape(2, EB, 128).transpose(1, 0, 2)
    q = edge_attr.reshape(EB, 128, 2, 8).transpose(2, 0, 3, 1)
    zeros_init = jnp.zeros((NPAD, DE), jnp.float32)
    partials = _sc_segment_sum(recv3, q, zeros_init)
    W1 = W[:DF]
    W2 = W[DF:]
    b2d = b.reshape(1, DF)
    return _tc_update(x, partials, W1, W2, b2d)
